# Initial kernel scaffold; baseline (speedup 1.0000x reference)
import jax
import jax.numpy as jnp
from jax.experimental import pallas as pl


def kernel(input_var, encoder_outputs, k, W_emb, W_h, W_out):
    raise NotImplementedError("write your pallas kernel here")


# speedup vs baseline: 5.0961x; 5.0961x over previous
import functools

import jax
import jax.numpy as jnp
from jax import lax
from jax.experimental import pallas as pl
from jax.experimental.pallas import tpu as pltpu

KB = 3
_NEG = -jnp.inf
_IMAX = 2**31 - 1


def _ctx_body(enc_ref, out_ref, acc_s, *, nc, inv):
    j = pl.program_id(1)

    @pl.when(j == 0)
    def _():
        acc_s[...] = jnp.zeros_like(acc_s)

    acc_s[...] += jnp.sum(enc_ref[...], axis=1)

    @pl.when(j == nc - 1)
    def _():
        out_ref[...] = acc_s[...] * inv


def _pooled_ctx(enc):
    b, seq, d = enc.shape
    bb, ch = 8, 256
    nc = seq // ch
    return pl.pallas_call(
        functools.partial(_ctx_body, nc=nc, inv=1.0 / seq),
        grid=(b // bb, nc),
        in_specs=[pl.BlockSpec((bb, ch, d), lambda i, j: (i, j, 0))],
        out_specs=pl.BlockSpec((bb, d), lambda i, j: (i, 0)),
        out_shape=jax.ShapeDtypeStruct((b, d), jnp.float32),
        scratch_shapes=[pltpu.VMEM((bb, d), jnp.float32)],
    )(enc)


def _step_body(emb_ref, ctx_ref, wh_ref, wout_ref, tv_ref, ti_ref, lse_ref,
               h_s, m_s, s_s, tv_s, ti_s, *, nt, vt, vocab):
    i = pl.program_id(0)

    @pl.when(i == 0)
    def _():
        x = emb_ref[...] + ctx_ref[...]
        h_s[...] = jnp.tanh(jnp.dot(x, wh_ref[...],
                                    preferred_element_type=jnp.float32))
        m_s[...] = jnp.full_like(m_s, _NEG)
        s_s[...] = jnp.zeros_like(s_s)
        tv_s[...] = jnp.full_like(tv_s, _NEG)
        ti_s[...] = jnp.zeros_like(ti_s)

    logits = jnp.dot(h_s[...], wout_ref[...],
                     preferred_element_type=jnp.float32)
    col = i * vt + lax.broadcasted_iota(jnp.int32, logits.shape, 1)
    masked = jnp.where(col < vocab, logits, _NEG)

    tile_m = jnp.max(masked, axis=1, keepdims=True)
    new_m = jnp.maximum(m_s[...], tile_m)
    s_s[...] = (s_s[...] * jnp.exp(m_s[...] - new_m)
                + jnp.sum(jnp.exp(masked - new_m), axis=1, keepdims=True))
    m_s[...] = new_m

    tv = tv_s[...]
    ti = ti_s[...]
    v1, v2, v3 = tv[:, 0:1], tv[:, 1:2], tv[:, 2:3]
    i1, i2, i3 = ti[:, 0:1], ti[:, 1:2], ti[:, 2:3]
    cur = masked
    for _ in range(KB):
        cm = jnp.max(cur, axis=1, keepdims=True)
        cidx = jnp.min(jnp.where(cur == cm, col, _IMAX), axis=1, keepdims=True)
        g1, g2, g3 = cm > v1, cm > v2, cm > v3
        v1, v2, v3 = (jnp.where(g1, cm, v1),
                      jnp.where(g1, v1, jnp.where(g2, cm, v2)),
                      jnp.where(g2, v2, jnp.where(g3, cm, v3)))
        i1, i2, i3 = (jnp.where(g1, cidx, i1),
                      jnp.where(g1, i1, jnp.where(g2, cidx, i2)),
                      jnp.where(g2, i2, jnp.where(g3, cidx, i3)))
        cur = jnp.where(col == cidx, _NEG, cur)
    tv_s[...] = jnp.concatenate([v1, v2, v3], axis=1)
    ti_s[...] = jnp.concatenate([i1, i2, i3], axis=1)

    @pl.when(i == nt - 1)
    def _():
        tv_ref[...] = tv_s[...]
        ti_ref[...] = ti_s[...]
        lse_ref[...] = m_s[...] + jnp.log(s_s[...])


def _fused_step(emb, ctx_rows, W_h, W_out, vt=2048):
    r, d = emb.shape
    vocab = W_out.shape[1]
    nt = pl.cdiv(vocab, vt)
    full = lambda i: (0, 0)
    return pl.pallas_call(
        functools.partial(_step_body, nt=nt, vt=vt, vocab=vocab),
        grid=(nt,),
        in_specs=[
            pl.BlockSpec((r, d), full),
            pl.BlockSpec((r, d), full),
            pl.BlockSpec((d, d), full),
            pl.BlockSpec((d, vt), lambda i: (0, i)),
        ],
        out_specs=[
            pl.BlockSpec((r, KB), full),
            pl.BlockSpec((r, KB), full),
            pl.BlockSpec((r, 1), full),
        ],
        out_shape=[
            jax.ShapeDtypeStruct((r, KB), jnp.float32),
            jax.ShapeDtypeStruct((r, KB), jnp.int32),
            jax.ShapeDtypeStruct((r, 1), jnp.float32),
        ],
        scratch_shapes=[
            pltpu.VMEM((r, d), jnp.float32),
            pltpu.VMEM((r, 1), jnp.float32),
            pltpu.VMEM((r, 1), jnp.float32),
            pltpu.VMEM((r, KB), jnp.float32),
            pltpu.VMEM((r, KB), jnp.int32),
        ],
    )(emb, ctx_rows, W_h, W_out)


def kernel(input_var, encoder_outputs, k, W_emb, W_h, W_out):
    bsz = encoder_outputs.shape[0]
    vocab = W_out.shape[1]

    ctx = _pooled_ctx(encoder_outputs)

    emb0 = jnp.take(W_emb, input_var[:, 0], axis=0)
    tv, ti, lse = _fused_step(emb0, ctx, W_h, W_out)
    cum_ps = tv - lse
    beams = ti[:, :, None]

    ctx_k = jnp.repeat(ctx, KB, axis=0)
    for _ in range(7):
        last = beams[:, :, -1].reshape(bsz * KB)
        emb = jnp.take(W_emb, last, axis=0)
        tv, ti, lse = _fused_step(emb, ctx_k, W_h, W_out)
        tv = tv.reshape(bsz, KB, KB)
        ti = ti.reshape(bsz, KB * KB)
        bias = cum_ps - lse.reshape(bsz, KB)
        cand = (tv + bias[:, :, None]).reshape(bsz, KB * KB)
        cum_ps, sel = lax.top_k(cand, KB)
        prev = sel // KB
        tok = jnp.take_along_axis(ti, sel, axis=1)
        beams = jnp.take_along_axis(beams, prev[:, :, None], axis=1)
        beams = jnp.concatenate([beams, tok[:, :, None]], axis=2)

    return beams[:, 0, :], cum_ps


# speedup vs baseline: 5.7010x; 1.1187x over previous
import functools

import jax
import jax.numpy as jnp
from jax import lax
from jax.experimental import pallas as pl
from jax.experimental.pallas import tpu as pltpu

KB = 3
_NEG = -jnp.inf
_IMAX = 2**31 - 1


def _ctx_body(enc_ref, out_ref, acc_s, *, nc, inv):
    j = pl.program_id(1)

    @pl.when(j == 0)
    def _():
        acc_s[...] = jnp.zeros_like(acc_s)

    acc_s[...] += jnp.sum(enc_ref[...], axis=1)

    @pl.when(j == nc - 1)
    def _():
        out_ref[...] = acc_s[...] * inv


def _pooled_ctx(enc):
    b, seq, d = enc.shape
    bb, ch = 8, 256
    nc = seq // ch
    return pl.pallas_call(
        functools.partial(_ctx_body, nc=nc, inv=1.0 / seq),
        grid=(b // bb, nc),
        in_specs=[pl.BlockSpec((bb, ch, d), lambda i, j: (i, j, 0))],
        out_specs=pl.BlockSpec((bb, d), lambda i, j: (i, 0)),
        out_shape=jax.ShapeDtypeStruct((b, d), jnp.float32),
        scratch_shapes=[pltpu.VMEM((bb, d), jnp.float32)],
    )(enc)


def _step_body(emb_ref, ctx_ref, wh_ref, wout_ref, tv_ref, ti_ref, lse_ref,
               h_s, m_s, s_s, tv_s, ti_s, *, nt, vt, vocab):
    i = pl.program_id(0)

    @pl.when(i == 0)
    def _():
        x = emb_ref[...] + ctx_ref[...]
        h_s[...] = jnp.tanh(jnp.dot(x, wh_ref[...],
                                    preferred_element_type=jnp.float32))
        m_s[...] = jnp.full_like(m_s, _NEG)
        s_s[...] = jnp.zeros_like(s_s)
        tv_s[...] = jnp.full_like(tv_s, _NEG)
        ti_s[...] = jnp.zeros_like(ti_s)

    def tile_update(mask_tail):
        logits = jnp.dot(h_s[...], wout_ref[...],
                         preferred_element_type=jnp.float32)
        colf = lax.broadcasted_iota(
            jnp.int32, logits.shape, 1).astype(jnp.float32)
        if mask_tail:
            masked = jnp.where(colf < vocab - (nt - 1) * vt, logits, _NEG)
        else:
            masked = logits
        ones = jnp.full((vt, 1), 1.0, jnp.float32)

        tile_m = jnp.max(masked, axis=1, keepdims=True)
        new_m = jnp.maximum(m_s[...], tile_m)
        e = jnp.exp(masked - new_m)
        s_s[...] = (s_s[...] * jnp.exp(m_s[...] - new_m)
                    + jnp.dot(e, ones, preferred_element_type=jnp.float32))
        m_s[...] = new_m

        tv = tv_s[...]
        ti = ti_s[...]
        v1, v2, v3 = tv[:, 0:1], tv[:, 1:2], tv[:, 2:3]
        i1, i2, i3 = ti[:, 0:1], ti[:, 1:2], ti[:, 2:3]
        cur = masked
        cm = tile_m
        for r in range(KB):
            eq = cur == cm
            cidxf = jnp.dot(jnp.where(eq, colf, 0.0), ones,
                            preferred_element_type=jnp.float32)
            cidx = i * vt + cidxf.astype(jnp.int32)
            g1, g2, g3 = cm > v1, cm > v2, cm > v3
            v1, v2, v3 = (jnp.where(g1, cm, v1),
                          jnp.where(g1, v1, jnp.where(g2, cm, v2)),
                          jnp.where(g2, v2, jnp.where(g3, cm, v3)))
            i1, i2, i3 = (jnp.where(g1, cidx, i1),
                          jnp.where(g1, i1, jnp.where(g2, cidx, i2)),
                          jnp.where(g2, i2, jnp.where(g3, cidx, i3)))
            if r < KB - 1:
                cur = jnp.where(eq, _NEG, cur)
                cm = jnp.max(cur, axis=1, keepdims=True)
        tv_s[...] = jnp.concatenate([v1, v2, v3], axis=1)
        ti_s[...] = jnp.concatenate([i1, i2, i3], axis=1)

    ragged = nt * vt != vocab
    if ragged:
        pl.when(i < nt - 1)(lambda: tile_update(False))
        pl.when(i == nt - 1)(lambda: tile_update(True))
    else:
        tile_update(False)

    @pl.when(i == nt - 1)
    def _():
        tv_ref[...] = tv_s[...]
        ti_ref[...] = ti_s[...]
        lse_ref[...] = m_s[...] + jnp.log(s_s[...])


def _fused_step(emb, ctx_rows, W_h, W_out, vt=2048):
    r, d = emb.shape
    vocab = W_out.shape[1]
    nt = pl.cdiv(vocab, vt)
    full = lambda i: (0, 0)
    return pl.pallas_call(
        functools.partial(_step_body, nt=nt, vt=vt, vocab=vocab),
        grid=(nt,),
        in_specs=[
            pl.BlockSpec((r, d), full),
            pl.BlockSpec((r, d), full),
            pl.BlockSpec((d, d), full),
            pl.BlockSpec((d, vt), lambda i: (0, i)),
        ],
        out_specs=[
            pl.BlockSpec((r, KB), full),
            pl.BlockSpec((r, KB), full),
            pl.BlockSpec((r, 1), full),
        ],
        out_shape=[
            jax.ShapeDtypeStruct((r, KB), jnp.float32),
            jax.ShapeDtypeStruct((r, KB), jnp.int32),
            jax.ShapeDtypeStruct((r, 1), jnp.float32),
        ],
        scratch_shapes=[
            pltpu.VMEM((r, d), jnp.float32),
            pltpu.VMEM((r, 1), jnp.float32),
            pltpu.VMEM((r, 1), jnp.float32),
            pltpu.VMEM((r, KB), jnp.float32),
            pltpu.VMEM((r, KB), jnp.int32),
        ],
    )(emb, ctx_rows, W_h, W_out)


def kernel(input_var, encoder_outputs, k, W_emb, W_h, W_out):
    bsz = encoder_outputs.shape[0]
    vocab = W_out.shape[1]

    ctx = _pooled_ctx(encoder_outputs)

    emb0 = jnp.take(W_emb, input_var[:, 0], axis=0)
    tv, ti, lse = _fused_step(emb0, ctx, W_h, W_out)
    cum_ps = tv - lse
    beams = ti[:, :, None]

    ctx_k = jnp.repeat(ctx, KB, axis=0)
    for _ in range(7):
        last = beams[:, :, -1].reshape(bsz * KB)
        emb = jnp.take(W_emb, last, axis=0)
        tv, ti, lse = _fused_step(emb, ctx_k, W_h, W_out)
        tv = tv.reshape(bsz, KB, KB)
        ti = ti.reshape(bsz, KB * KB)
        bias = cum_ps - lse.reshape(bsz, KB)
        cand = (tv + bias[:, :, None]).reshape(bsz, KB * KB)
        cum_ps, sel = lax.top_k(cand, KB)
        prev = sel // KB
        tok = jnp.take_along_axis(ti, sel, axis=1)
        beams = jnp.take_along_axis(beams, prev[:, :, None], axis=1)
        beams = jnp.concatenate([beams, tok[:, :, None]], axis=2)

    return beams[:, 0, :], cum_ps
